# baseline (speedup 1.0000x reference)
import functools

import jax
import jax.numpy as jnp
from jax import lax
from jax.experimental import pallas as pl
from jax.experimental.pallas import tpu as pltpu
from jax.experimental.pallas import tpu_sc as plsc

NUM_CHARTS = 8
LATENT_DIM = 1024
RANK = 512
B = 2048
T = 256
NT = B // T
NI = NT + NUM_CHARTS - 1
NW = 32
BPW = B // NW


def _sc_permute(table, idx, scatter):
    D = table.shape[1]
    mesh = plsc.VectorSubcoreMesh(core_axis_name="c", subcore_axis_name="s")

    @functools.partial(
        pl.kernel,
        out_type=jax.ShapeDtypeStruct((B, D), table.dtype),
        mesh=mesh,
        scratch_types=[
            pltpu.VMEM((BPW,), jnp.int32),
            pltpu.VMEM((BPW, D), table.dtype),
            pltpu.SemaphoreType.DMA,
        ],
    )
    def gk(table_hbm, idx_hbm, out_hbm, idx_v, rows_v, sem):
        wid = lax.axis_index("s") * 2 + lax.axis_index("c")
        base = wid * BPW
        pltpu.sync_copy(idx_hbm.at[pl.ds(base, BPW)], idx_v)
        if scatter:
            pltpu.sync_copy(table_hbm.at[pl.ds(base, BPW)], rows_v)
            pltpu.async_copy(rows_v, out_hbm.at[idx_v], sem).wait()
        else:
            pltpu.async_copy(table_hbm.at[idx_v], rows_v, sem).wait()
            pltpu.sync_copy(rows_v, out_hbm.at[pl.ds(base, BPW)])

    return gk(table, idx)


def _sc_regroup(table, src_idx, dst_idx):
    D = table.shape[1]
    mesh = plsc.VectorSubcoreMesh(core_axis_name="c", subcore_axis_name="s")

    @functools.partial(
        pl.kernel,
        out_type=jax.ShapeDtypeStruct((B, D), table.dtype),
        mesh=mesh,
        scratch_types=[
            pltpu.VMEM((BPW,), jnp.int32),
            pltpu.VMEM((BPW,), jnp.int32),
            pltpu.VMEM((BPW, D), table.dtype),
            pltpu.SemaphoreType.DMA,
            pltpu.SemaphoreType.DMA,
        ],
    )
    def gk(table_hbm, sidx_hbm, didx_hbm, out_hbm, sidx_v, didx_v, rows_v, sem1, sem2):
        wid = lax.axis_index("s") * 2 + lax.axis_index("c")
        base = wid * BPW
        pltpu.sync_copy(sidx_hbm.at[pl.ds(base, BPW)], sidx_v)
        pltpu.sync_copy(didx_hbm.at[pl.ds(base, BPW)], didx_v)
        pltpu.async_copy(table_hbm.at[sidx_v], rows_v, sem1).wait()
        pltpu.async_copy(rows_v, out_hbm.at[didx_v], sem2).wait()

    return gk(table, src_idx, dst_idx)


def _routing2(ids2):
    eye = jnp.arange(NUM_CHARTS, dtype=jnp.int32)[None, None, :]
    oh = (ids2[:, :, None] == eye).astype(jnp.int32)
    csum = jnp.cumsum(oh, axis=1)
    counts = csum[:, -1, :]
    off = jnp.concatenate(
        [jnp.zeros((2, 1), jnp.int32),
         jnp.cumsum(counts, axis=1).astype(jnp.int32)], axis=1)
    rank = jnp.sum((csum - 1) * oh, axis=2)
    base = jnp.sum(off[:, None, :NUM_CHARTS] * oh, axis=2)
    inv = (base + rank).astype(jnp.int32)

    ft = off[:, :-1] // T
    lt = (off[:, 1:] - 1) // T
    n_items = jnp.where(counts > 0, lt - ft + 1, 0)
    start = jnp.concatenate(
        [jnp.zeros((2, 1), jnp.int32),
         jnp.cumsum(n_items, axis=1).astype(jnp.int32)], axis=1)
    total = start[:, -1:]
    g = jnp.arange(NI, dtype=jnp.int32)[None, :]
    e = jnp.sum((start[:, :, None] <= g[:, None, :]).astype(jnp.int32),
                axis=1) - 1
    e = jnp.clip(e, 0, NUM_CHARTS - 1).astype(jnp.int32)
    tile = jnp.take_along_axis(ft, e, axis=1) + (
        g - jnp.take_along_axis(start[:, :NUM_CHARTS], e, axis=1))
    valid = g < total
    tile = jnp.where(valid, tile, NT - 1).astype(jnp.int32)
    last_e = jnp.max(jnp.where(valid, e, -1), axis=1, keepdims=True)
    e = jnp.where(valid, e, last_e).astype(jnp.int32)
    lo = jnp.clip(jnp.take_along_axis(off, e, axis=1) - tile * T, 0, T)
    hi = jnp.clip(jnp.take_along_axis(off, e + 1, axis=1) - tile * T, 0, T)
    lo = jnp.where(valid, lo, 0).astype(jnp.int32)
    hi = jnp.where(valid, hi, 0).astype(jnp.int32)
    first = jnp.concatenate(
        [jnp.ones((2, 1), jnp.int32),
         (tile[:, 1:] != tile[:, :-1]).astype(jnp.int32)], axis=1)
    meta = jnp.concatenate([tile, e, lo, hi, first], axis=1).astype(jnp.int32)
    return inv, meta


def _gmm_body(out_dtype, meta_ref, x_ref, w_ref, bias_ref, out_ref):
    g = pl.program_id(0)
    lo = meta_ref[2 * NI + g]
    hi = meta_ref[3 * NI + g]
    first = meta_ref[4 * NI + g]

    @pl.when(lo < hi)
    def _():
        rowid = lax.broadcasted_iota(jnp.int32, (T, 1), 0)
        mask = (rowid >= lo) & (rowid < hi)
        xb = x_ref[...].astype(jnp.bfloat16)
        wb = w_ref[0].astype(jnp.bfloat16)
        val = lax.dot_general(xb, wb, (((1,), (1,)), ((), ())),
                              preferred_element_type=jnp.float32)
        val = (val + bias_ref[0]).astype(out_dtype)

        @pl.when(first == 1)
        def _():
            out_ref[...] = jnp.where(mask, val, 0)

        @pl.when(first == 0)
        def _():
            out_ref[...] = jnp.where(mask, val, out_ref[...])


def _grouped_matmul(meta, x, w, bias, n_in, n_out, out_dtype):
    return pl.pallas_call(
        functools.partial(_gmm_body, out_dtype),
        grid_spec=pltpu.PrefetchScalarGridSpec(
            num_scalar_prefetch=1,
            grid=(NI,),
            in_specs=[
                pl.BlockSpec((T, n_in), lambda g, m: (m[g], 0)),
                pl.BlockSpec((1, n_out, n_in), lambda g, m: (m[NI + g], 0, 0)),
                pl.BlockSpec((1, 1, n_out), lambda g, m: (m[NI + g], 0, 0)),
            ],
            out_specs=pl.BlockSpec((T, n_out), lambda g, m: (m[g], 0)),
        ),
        out_shape=jax.ShapeDtypeStruct((B, n_out), out_dtype),
    )(meta, x, w, bias.reshape(NUM_CHARTS, 1, n_out))


@jax.jit
def kernel(z_n, source_idx, target_idx, W_enc, W_dec, c, d):
    ids2 = jnp.stack([source_idx.astype(jnp.int32),
                      target_idx.astype(jnp.int32)])
    inv, meta = _routing2(ids2)
    inv_s, inv_t = inv[0], inv[1]

    z_s = _sc_permute(z_n, inv_s, scatter=True)
    h_s = _grouped_matmul(meta[0], z_s, W_enc, c, LATENT_DIM, RANK,
                          jnp.float32)
    h_t = _sc_regroup(h_s, inv_s, inv_t)
    y_t = _grouped_matmul(meta[1], h_t, W_dec, d, RANK, LATENT_DIM,
                          jnp.float32)
    return _sc_permute(y_t, inv_t, scatter=False)


# speedup vs baseline: 2.4013x; 2.4013x over previous
import jax
import jax.numpy as jnp
from jax import lax
from jax.experimental import pallas as pl

NUM_CHARTS = 8
LATENT_DIM = 1024
RANK = 512
B = 2048
T = 256
NT = B // T


def _body(z_ref, s_ref, t_ref, we_ref, wd_ref, c_ref, d_ref, out_ref):
    zb = z_ref[...].astype(jnp.bfloat16)
    sid = s_ref[...]
    tid = t_ref[...]
    lane8 = lax.broadcasted_iota(jnp.int32, (T, NUM_CHARTS), 1)
    oh_s = (sid == lane8)
    oh_t = (tid == lane8)

    h = jnp.zeros((T, RANK), jnp.float32)
    for e in range(NUM_CHARTS):
        part = lax.dot_general(zb, we_ref[e], (((1,), (1,)), ((), ())),
                               preferred_element_type=jnp.float32)
        h = jnp.where(oh_s[:, e:e + 1], part, h)
    h = h + lax.dot_general(oh_s.astype(jnp.float32), c_ref[...],
                            (((1,), (0,)), ((), ())),
                            preferred_element_type=jnp.float32)

    hb = h.astype(jnp.bfloat16)
    y = jnp.zeros((T, LATENT_DIM), jnp.float32)
    for e in range(NUM_CHARTS):
        part = lax.dot_general(hb, wd_ref[e], (((1,), (1,)), ((), ())),
                               preferred_element_type=jnp.float32)
        y = jnp.where(oh_t[:, e:e + 1], part, y)
    y = y + lax.dot_general(oh_t.astype(jnp.float32), d_ref[...],
                            (((1,), (0,)), ((), ())),
                            preferred_element_type=jnp.float32)
    out_ref[...] = y


@jax.jit
def kernel(z_n, source_idx, target_idx, W_enc, W_dec, c, d):
    src = source_idx.astype(jnp.int32).reshape(B, 1)
    tgt = target_idx.astype(jnp.int32).reshape(B, 1)
    return pl.pallas_call(
        _body,
        grid=(NT,),
        in_specs=[
            pl.BlockSpec((T, LATENT_DIM), lambda i: (i, 0)),
            pl.BlockSpec((T, 1), lambda i: (i, 0)),
            pl.BlockSpec((T, 1), lambda i: (i, 0)),
            pl.BlockSpec((NUM_CHARTS, RANK, LATENT_DIM), lambda i: (0, 0, 0)),
            pl.BlockSpec((NUM_CHARTS, LATENT_DIM, RANK), lambda i: (0, 0, 0)),
            pl.BlockSpec((NUM_CHARTS, RANK), lambda i: (0, 0)),
            pl.BlockSpec((NUM_CHARTS, LATENT_DIM), lambda i: (0, 0)),
        ],
        out_specs=pl.BlockSpec((T, LATENT_DIM), lambda i: (i, 0)),
        out_shape=jax.ShapeDtypeStruct((B, LATENT_DIM), jnp.float32),
    )(z_n, src, tgt, W_enc.astype(jnp.bfloat16), W_dec.astype(jnp.bfloat16),
      c, d)


# speedup vs baseline: 2.9317x; 1.2209x over previous
import jax
import jax.numpy as jnp
from jax import lax
from jax.experimental import pallas as pl

NUM_CHARTS = 8
LATENT_DIM = 1024
RANK = 512
B = 2048
T = 256
NT = B // T


def _body(z_ref, s_ref, t_ref, we_ref, wd_ref, c_ref, d_ref, out_ref):
    zb = z_ref[...]
    sid = s_ref[...]
    tid = t_ref[...]
    lane8 = lax.broadcasted_iota(jnp.int32, (T, NUM_CHARTS), 1)
    oh_s = (sid == lane8)
    oh_t = (tid == lane8)

    h = jnp.zeros((T, RANK), jnp.float32)
    for e in range(NUM_CHARTS):
        part = lax.dot_general(zb, we_ref[e], (((1,), (1,)), ((), ())),
                               preferred_element_type=jnp.float32)
        h = jnp.where(oh_s[:, e:e + 1], part, h)
    h = h + lax.dot_general(oh_s.astype(jnp.float32), c_ref[...],
                            (((1,), (0,)), ((), ())),
                            preferred_element_type=jnp.float32)

    hb = h
    y = jnp.zeros((T, LATENT_DIM), jnp.float32)
    for e in range(NUM_CHARTS):
        part = lax.dot_general(hb, wd_ref[e], (((1,), (1,)), ((), ())),
                               preferred_element_type=jnp.float32)
        y = jnp.where(oh_t[:, e:e + 1], part, y)
    y = y + lax.dot_general(oh_t.astype(jnp.float32), d_ref[...],
                            (((1,), (0,)), ((), ())),
                            preferred_element_type=jnp.float32)
    out_ref[...] = y


@jax.jit
def kernel(z_n, source_idx, target_idx, W_enc, W_dec, c, d):
    src = source_idx.astype(jnp.int32).reshape(B, 1)
    tgt = target_idx.astype(jnp.int32).reshape(B, 1)
    return pl.pallas_call(
        _body,
        grid=(NT,),
        in_specs=[
            pl.BlockSpec((T, LATENT_DIM), lambda i: (i, 0)),
            pl.BlockSpec((T, 1), lambda i: (i, 0)),
            pl.BlockSpec((T, 1), lambda i: (i, 0)),
            pl.BlockSpec((NUM_CHARTS, RANK, LATENT_DIM), lambda i: (0, 0, 0)),
            pl.BlockSpec((NUM_CHARTS, LATENT_DIM, RANK), lambda i: (0, 0, 0)),
            pl.BlockSpec((NUM_CHARTS, RANK), lambda i: (0, 0)),
            pl.BlockSpec((NUM_CHARTS, LATENT_DIM), lambda i: (0, 0)),
        ],
        out_specs=pl.BlockSpec((T, LATENT_DIM), lambda i: (i, 0)),
        out_shape=jax.ShapeDtypeStruct((B, LATENT_DIM), jnp.float32),
    )(z_n, src, tgt, W_enc, W_dec,
      c, d)


# speedup vs baseline: 3.0402x; 1.0370x over previous
import jax
import jax.numpy as jnp
from jax import lax
from jax.experimental import pallas as pl

NUM_CHARTS = 8
LATENT_DIM = 1024
RANK = 512
B = 2048
T = 512
NT = B // T


def _body(z_ref, s_ref, t_ref, we_ref, wd_ref, c_ref, d_ref, out_ref):
    zb = z_ref[...]
    sid = s_ref[...]
    tid = t_ref[...]
    lane8 = lax.broadcasted_iota(jnp.int32, (T, NUM_CHARTS), 1)
    oh_s = (sid == lane8)
    oh_t = (tid == lane8)

    h = jnp.zeros((T, RANK), jnp.float32)
    for e in range(NUM_CHARTS):
        part = lax.dot_general(zb, we_ref[e], (((1,), (1,)), ((), ())),
                               preferred_element_type=jnp.float32)
        h = jnp.where(oh_s[:, e:e + 1], part, h)
    h = h + lax.dot_general(oh_s.astype(jnp.float32), c_ref[...],
                            (((1,), (0,)), ((), ())),
                            preferred_element_type=jnp.float32)

    hb = h
    y = jnp.zeros((T, LATENT_DIM), jnp.float32)
    for e in range(NUM_CHARTS):
        part = lax.dot_general(hb, wd_ref[e], (((1,), (1,)), ((), ())),
                               preferred_element_type=jnp.float32)
        y = jnp.where(oh_t[:, e:e + 1], part, y)
    y = y + lax.dot_general(oh_t.astype(jnp.float32), d_ref[...],
                            (((1,), (0,)), ((), ())),
                            preferred_element_type=jnp.float32)
    out_ref[...] = y


@jax.jit
def kernel(z_n, source_idx, target_idx, W_enc, W_dec, c, d):
    src = source_idx.astype(jnp.int32).reshape(B, 1)
    tgt = target_idx.astype(jnp.int32).reshape(B, 1)
    return pl.pallas_call(
        _body,
        grid=(NT,),
        in_specs=[
            pl.BlockSpec((T, LATENT_DIM), lambda i: (i, 0)),
            pl.BlockSpec((T, 1), lambda i: (i, 0)),
            pl.BlockSpec((T, 1), lambda i: (i, 0)),
            pl.BlockSpec((NUM_CHARTS, RANK, LATENT_DIM), lambda i: (0, 0, 0)),
            pl.BlockSpec((NUM_CHARTS, LATENT_DIM, RANK), lambda i: (0, 0, 0)),
            pl.BlockSpec((NUM_CHARTS, RANK), lambda i: (0, 0)),
            pl.BlockSpec((NUM_CHARTS, LATENT_DIM), lambda i: (0, 0)),
        ],
        out_specs=pl.BlockSpec((T, LATENT_DIM), lambda i: (i, 0)),
        out_shape=jax.ShapeDtypeStruct((B, LATENT_DIM), jnp.float32),
    )(z_n, src, tgt, W_enc, W_dec,
      c, d)
